# baseline (speedup 1.0000x reference)
import functools

import jax
import jax.numpy as jnp
from jax import lax
from jax.experimental import pallas as pl
from jax.experimental.pallas import tpu as pltpu
from jax.experimental.pallas import tpu_sc as plsc

N = 10000
E = 320000
NC = 2
NS = 16
NW = NC * NS
EPW = E // NW
KB = 80
NB = EPW // KB
SLAB = 632


def _slab_base(sid):
    return jnp.minimum(sid * SLAB, N - SLAB)

_MESH = plsc.VectorSubcoreMesh(core_axis_name="c", subcore_axis_name="s")
_SC_PARAMS = pltpu.CompilerParams(use_tc_tiling_on_sc=False)


NBUF = 5
NR = NB // NBUF


def _deg_body(dst_hbm, w_hbm, z_hbm, deg_hbm, dstp, wp, degsh, *sems):
    cid = lax.axis_index("c")
    sid = lax.axis_index("s")
    wid = cid * NS + sid
    sb = _slab_base(sid)
    pltpu.sync_copy(z_hbm.at[pl.ds(sb, SLAB)], degsh.at[pl.ds(sb, SLAB)])
    pltpu.sync_copy(dst_hbm.at[wid], dstp)
    pltpu.sync_copy(w_hbm.at[wid], wp)
    plsc.subcore_barrier()

    def round_(r, carry):
        base = r * NBUF
        for b in range(NBUF):
            pltpu.async_copy(wp.at[base + b], degsh.at[dstp.at[base + b]],
                             sems[b], add=True)
        for b in range(NBUF):
            pltpu.make_async_copy(wp.at[base + b], degsh.at[dstp.at[base + b]],
                                  sems[b]).wait()
        return carry

    lax.fori_loop(0, NR, round_, 0)
    plsc.subcore_barrier()
    pltpu.sync_copy(degsh.at[pl.ds(sb, SLAB)],
                    deg_hbm.at[pl.ds(cid * N + sb, SLAB)])


_deg_kernel = pl.kernel(
    _deg_body,
    out_type=jax.ShapeDtypeStruct((2 * N, 1), jnp.float32),
    mesh=_MESH,
    compiler_params=_SC_PARAMS,
    scratch_types=[
        pltpu.VMEM((NB, KB), jnp.int32),
        pltpu.VMEM((NB, KB, 1), jnp.float32),
        pltpu.VMEM_SHARED((N, 1), jnp.float32),
    ] + [pltpu.SemaphoreType.DMA] * NBUF,
)


FH = 64


def _msg_pass(ht_hbm, out_hbm, srcp, dstp, wp, rows, accsh, gsem, ssem,
              cid, sid, sb):
    def zrow(k, c):
        for f in range(FH // 16):
            rows[0, k, pl.ds(f * 16, 16)] = jnp.zeros((16,), jnp.float32)
        return c

    lax.fori_loop(0, KB, zrow, 0)
    for j in range(SLAB // KB):
        pltpu.sync_copy(rows.at[0], accsh.at[pl.ds(sb + j * KB, KB)])
    _tail = SLAB % KB
    pltpu.sync_copy(rows.at[0, pl.ds(0, _tail)],
                    accsh.at[pl.ds(sb + (SLAB // KB) * KB, _tail)])
    plsc.subcore_barrier()

    for b in range(NBUF):
        pltpu.async_copy(ht_hbm.at[srcp.at[b]], rows.at[b], gsem[b])

    def round_(r, carry):
        base = r * NBUF
        for b in range(NBUF):
            row = base + b
            pltpu.make_async_copy(ht_hbm.at[srcp.at[row]], rows.at[b],
                                  gsem[b]).wait()

            def scale16(g16, c):
                wv = wp[row, pl.ds(g16 * 16, 16)]
                for j in range(16):
                    k = g16 * 16 + j
                    s = wv[j]
                    for f in range(FH // 16):
                        sl = pl.ds(f * 16, 16)
                        rows[b, k, sl] = rows[b, k, sl] * s
                return c

            lax.fori_loop(0, KB // 16, scale16, 0)
            pltpu.async_copy(rows.at[b], accsh.at[dstp.at[row]], ssem[b],
                             add=True)
        for b in range(NBUF):
            pltpu.make_async_copy(rows.at[b], accsh.at[dstp.at[base + b]],
                                  ssem[b]).wait()

        @pl.when(r < NR - 1)
        def _():
            for b in range(NBUF):
                nxt = base + NBUF + b
                pltpu.async_copy(ht_hbm.at[srcp.at[nxt]], rows.at[b], gsem[b])

        return carry

    lax.fori_loop(0, NR, round_, 0)
    plsc.subcore_barrier()
    pltpu.sync_copy(accsh.at[pl.ds(sb, SLAB)],
                    out_hbm.at[pl.ds(cid * N + sb, SLAB)])


def _msg1_body(src_hbm, dst_hbm, w_hbm, hta_hbm, htb_hbm, outa_hbm, outb_hbm,
               srcp, dstp, wp, rows, accsh, *sems):
    gsem, ssem = sems[:NBUF], sems[NBUF:]
    cid = lax.axis_index("c")
    sid = lax.axis_index("s")
    wid = cid * NS + sid
    sb = _slab_base(sid)
    pltpu.sync_copy(src_hbm.at[wid], srcp)
    pltpu.sync_copy(dst_hbm.at[wid], dstp)
    pltpu.sync_copy(w_hbm.at[wid], wp)
    _msg_pass(hta_hbm, outa_hbm, srcp, dstp, wp, rows, accsh, gsem, ssem,
              cid, sid, sb)
    _msg_pass(htb_hbm, outb_hbm, srcp, dstp, wp, rows, accsh, gsem, ssem,
              cid, sid, sb)


def _msg2_body(src_hbm, dst_hbm, w_hbm, ht_hbm, out_hbm,
               srcp, dstp, wp, rows, accsh, *sems):
    gsem, ssem = sems[:NBUF], sems[NBUF:]
    cid = lax.axis_index("c")
    sid = lax.axis_index("s")
    wid = cid * NS + sid
    sb = _slab_base(sid)
    pltpu.sync_copy(src_hbm.at[wid], srcp)
    pltpu.sync_copy(dst_hbm.at[wid], dstp)
    pltpu.sync_copy(w_hbm.at[wid], wp)
    _msg_pass(ht_hbm, out_hbm, srcp, dstp, wp, rows, accsh, gsem, ssem,
              cid, sid, sb)


_MSG_SCRATCH = [
    pltpu.VMEM((NB, KB), jnp.int32),
    pltpu.VMEM((NB, KB), jnp.int32),
    pltpu.VMEM((NB, KB), jnp.float32),
    pltpu.VMEM((NBUF, KB, FH), jnp.float32),
    pltpu.VMEM_SHARED((N, FH), jnp.float32),
] + [pltpu.SemaphoreType.DMA] * (2 * NBUF)

_msg_kernel_128 = pl.kernel(
    _msg1_body,
    out_type=[jax.ShapeDtypeStruct((2 * N, FH), jnp.float32),
              jax.ShapeDtypeStruct((2 * N, FH), jnp.float32)],
    mesh=_MESH,
    compiler_params=_SC_PARAMS,
    scratch_types=_MSG_SCRATCH,
)

_msg_kernel_64 = pl.kernel(
    _msg2_body,
    out_type=jax.ShapeDtypeStruct((2 * N, FH), jnp.float32),
    mesh=_MESH,
    compiler_params=_SC_PARAMS,
    scratch_types=_MSG_SCRATCH,
)


_R = 1000
_G = N // _R


def _tc1_body(x_ref, w_ref, dga_ref, dgb_ref, hta_ref, htb_ref, dis_ref):
    dis = lax.rsqrt(dga_ref[...] + dgb_ref[...] + 1.0)
    ht = jnp.dot(x_ref[...], w_ref[...],
                 preferred_element_type=jnp.float32) * dis
    hta_ref[...] = ht[:, :FH]
    htb_ref[...] = ht[:, FH:]
    dis_ref[...] = dis


def _tc1(x, W1, deg_parts):
    return pl.pallas_call(
        _tc1_body,
        grid=(_G,),
        in_specs=[
            pl.BlockSpec((_R, 128), lambda i: (i, 0)),
            pl.BlockSpec((128, 128), lambda i: (0, 0)),
            pl.BlockSpec((_R, 1), lambda i: (i, 0)),
            pl.BlockSpec((_R, 1), lambda i: (i + _G, 0)),
        ],
        out_specs=[
            pl.BlockSpec((_R, FH), lambda i: (i, 0)),
            pl.BlockSpec((_R, FH), lambda i: (i, 0)),
            pl.BlockSpec((_R, 1), lambda i: (i, 0)),
        ],
        out_shape=[
            jax.ShapeDtypeStruct((N, FH), jnp.float32),
            jax.ShapeDtypeStruct((N, FH), jnp.float32),
            jax.ShapeDtypeStruct((N, 1), jnp.float32),
        ],
    )(x, W1, deg_parts, deg_parts)


def _tc2_body(a0a_ref, a1a_ref, a0b_ref, a1b_ref, hta_ref, htb_ref,
              dis_ref, b_ref, w_ref, out_ref):
    dis = dis_ref[...]
    xa = dis * (a0a_ref[...] + a1a_ref[...] + hta_ref[...])
    xb = dis * (a0b_ref[...] + a1b_ref[...] + htb_ref[...])
    x1 = jnp.concatenate([xa, xb], axis=1) + b_ref[...]
    out_ref[...] = jnp.dot(x1, w_ref[...],
                           preferred_element_type=jnp.float32) * dis


def _tc2(acc1a, acc1b, ht1a, ht1b, dis, b1, W2):
    return pl.pallas_call(
        _tc2_body,
        grid=(_G,),
        in_specs=[
            pl.BlockSpec((_R, FH), lambda i: (i, 0)),
            pl.BlockSpec((_R, FH), lambda i: (i + _G, 0)),
            pl.BlockSpec((_R, FH), lambda i: (i, 0)),
            pl.BlockSpec((_R, FH), lambda i: (i + _G, 0)),
            pl.BlockSpec((_R, FH), lambda i: (i, 0)),
            pl.BlockSpec((_R, FH), lambda i: (i, 0)),
            pl.BlockSpec((_R, 1), lambda i: (i, 0)),
            pl.BlockSpec((1, 128), lambda i: (0, 0)),
            pl.BlockSpec((128, 64), lambda i: (0, 0)),
        ],
        out_specs=pl.BlockSpec((_R, 64), lambda i: (i, 0)),
        out_shape=jax.ShapeDtypeStruct((N, 64), jnp.float32),
    )(acc1a, acc1a, acc1b, acc1b, ht1a, ht1b, dis, b1, W2)


def _tc3_body(a0_ref, a1_ref, ht_ref, dis_ref, b_ref, out_ref):
    out_ref[...] = (dis_ref[...] * (a0_ref[...] + a1_ref[...] + ht_ref[...])
                    + b_ref[...])


def _tc3(acc2, ht2, dis, b2):
    return pl.pallas_call(
        _tc3_body,
        grid=(_G,),
        in_specs=[
            pl.BlockSpec((_R, 64), lambda i: (i, 0)),
            pl.BlockSpec((_R, 64), lambda i: (i + _G, 0)),
            pl.BlockSpec((_R, 64), lambda i: (i, 0)),
            pl.BlockSpec((_R, 1), lambda i: (i, 0)),
            pl.BlockSpec((1, 64), lambda i: (0, 0)),
        ],
        out_specs=pl.BlockSpec((_R, 64), lambda i: (i, 0)),
        out_shape=jax.ShapeDtypeStruct((N, 64), jnp.float32),
    )(acc2, acc2, ht2, dis, b2)


def kernel(x, edge_index, edge_weight, W1, b1, W2, b2):
    src = edge_index[0].astype(jnp.int32).reshape(NW, NB, KB)
    dst = edge_index[1].astype(jnp.int32).reshape(NW, NB, KB)
    w = edge_weight.astype(jnp.float32).reshape(NW, NB, KB)
    w4 = w.reshape(NW, NB, KB, 1)
    z1 = jnp.zeros((N, 1), jnp.float32)

    deg_parts = _deg_kernel(dst, w4, z1)
    ht1a, ht1b, dis = _tc1(x, W1, deg_parts)
    acc1a, acc1b = _msg_kernel_128(src, dst, w, ht1a, ht1b)
    ht2 = _tc2(acc1a, acc1b, ht1a, ht1b, dis,
               b1.reshape(1, -1), W2)
    acc2 = _msg_kernel_64(src, dst, w, ht2)
    return _tc3(acc2, ht2, dis, b2.reshape(1, -1))


# speedup vs baseline: 1.6173x; 1.6173x over previous
import functools

import jax
import jax.numpy as jnp
from jax import lax
from jax.experimental import pallas as pl
from jax.experimental.pallas import tpu as pltpu
from jax.experimental.pallas import tpu_sc as plsc

N = 10000
E = 320000
NC = 2
NS = 16
NW = NC * NS
EPW = E // NW
KB = 80
NB = EPW // KB
SLAB = 632


def _slab_base(sid):
    return jnp.minimum(sid * SLAB, N - SLAB)

_MESH = plsc.VectorSubcoreMesh(core_axis_name="c", subcore_axis_name="s")
_SC_PARAMS = pltpu.CompilerParams(use_tc_tiling_on_sc=False)


NBUF = 5
NR = NB // NBUF


def _deg_body(dst_hbm, w_hbm, z_hbm, deg_hbm, dstp, wp, rows, degsh, *sems):
    cid = lax.axis_index("c")
    sid = lax.axis_index("s")
    wid = cid * NS + sid
    sb = _slab_base(sid)
    pltpu.sync_copy(z_hbm.at[pl.ds(sb, SLAB)], degsh.at[pl.ds(sb, SLAB)])
    pltpu.sync_copy(dst_hbm.at[pl.ds(wid * EPW, EPW)], dstp)
    pltpu.sync_copy(w_hbm.at[pl.ds(wid * EPW, EPW)], wp)
    plsc.subcore_barrier()

    def round_(r, carry):
        base = r * NBUF
        for b in range(NBUF):
            e0 = (base + b) * KB
            for g16 in range(KB // 16):
                wv = wp[pl.ds(e0 + g16 * 16, 16)]
                for j in range(16):
                    rows[b, g16 * 16 + j, :] = jnp.broadcast_to(wv[j], (16,))
            pltpu.async_copy(rows.at[b], degsh.at[dstp.at[pl.ds(e0, KB)]],
                             sems[b], add=True)
        for b in range(NBUF):
            e0 = (base + b) * KB
            pltpu.make_async_copy(rows.at[b], degsh.at[dstp.at[pl.ds(e0, KB)]],
                                  sems[b]).wait()
        return carry

    lax.fori_loop(0, NR, round_, 0)
    plsc.subcore_barrier()
    pltpu.sync_copy(degsh.at[pl.ds(sb, SLAB)],
                    deg_hbm.at[pl.ds(cid * N + sb, SLAB)])


_deg_kernel = pl.kernel(
    _deg_body,
    out_type=jax.ShapeDtypeStruct((2 * N, 16), jnp.float32),
    mesh=_MESH,
    compiler_params=_SC_PARAMS,
    scratch_types=[
        pltpu.VMEM((EPW,), jnp.int32),
        pltpu.VMEM((EPW,), jnp.float32),
        pltpu.VMEM((NBUF, KB, 16), jnp.float32),
        pltpu.VMEM_SHARED((N, 16), jnp.float32),
    ] + [pltpu.SemaphoreType.DMA] * NBUF,
)


FH = 64


def _msg_pass(ht_hbm, out_hbm, srcp, dstp, wp, rows, accsh, gsem, ssem,
              cid, sid, sb):
    def zrow(k, c):
        for f in range(FH // 16):
            rows[0, k, pl.ds(f * 16, 16)] = jnp.zeros((16,), jnp.float32)
        return c

    lax.fori_loop(0, KB, zrow, 0)
    for j in range(SLAB // KB):
        pltpu.sync_copy(rows.at[0], accsh.at[pl.ds(sb + j * KB, KB)])
    _tail = SLAB % KB
    pltpu.sync_copy(rows.at[0, pl.ds(0, _tail)],
                    accsh.at[pl.ds(sb + (SLAB // KB) * KB, _tail)])
    plsc.subcore_barrier()

    for b in range(NBUF):
        pltpu.async_copy(ht_hbm.at[srcp.at[pl.ds(b * KB, KB)]], rows.at[b],
                         gsem[b])

    def round_(r, carry):
        base = r * NBUF
        for b in range(NBUF):
            e0 = (base + b) * KB
            pltpu.make_async_copy(ht_hbm.at[srcp.at[pl.ds(e0, KB)]],
                                  rows.at[b], gsem[b]).wait()

            def scale16(g16, c):
                wv = wp[pl.ds(e0 + g16 * 16, 16)]
                for j in range(16):
                    k = g16 * 16 + j
                    s = wv[j]
                    for f in range(FH // 16):
                        sl = pl.ds(f * 16, 16)
                        rows[b, k, sl] = rows[b, k, sl] * s
                return c

            lax.fori_loop(0, KB // 16, scale16, 0)
            pltpu.async_copy(rows.at[b], accsh.at[dstp.at[pl.ds(e0, KB)]],
                             ssem[b], add=True)
        for b in range(NBUF):
            e0 = (base + b) * KB
            pltpu.make_async_copy(rows.at[b], accsh.at[dstp.at[pl.ds(e0, KB)]],
                                  ssem[b]).wait()

        @pl.when(r < NR - 1)
        def _():
            for b in range(NBUF):
                e0 = (base + NBUF + b) * KB
                pltpu.async_copy(ht_hbm.at[srcp.at[pl.ds(e0, KB)]],
                                 rows.at[b], gsem[b])

        return carry

    lax.fori_loop(0, NR, round_, 0)
    plsc.subcore_barrier()
    pltpu.sync_copy(accsh.at[pl.ds(sb, SLAB)],
                    out_hbm.at[pl.ds(cid * N + sb, SLAB)])


def _msg1_body(src_hbm, dst_hbm, w_hbm, hta_hbm, htb_hbm, outa_hbm, outb_hbm,
               srcp, dstp, wp, rows, accsh, *sems):
    gsem, ssem = sems[:NBUF], sems[NBUF:]
    cid = lax.axis_index("c")
    sid = lax.axis_index("s")
    wid = cid * NS + sid
    sb = _slab_base(sid)
    pltpu.sync_copy(src_hbm.at[pl.ds(wid * EPW, EPW)], srcp)
    pltpu.sync_copy(dst_hbm.at[pl.ds(wid * EPW, EPW)], dstp)
    pltpu.sync_copy(w_hbm.at[pl.ds(wid * EPW, EPW)], wp)
    _msg_pass(hta_hbm, outa_hbm, srcp, dstp, wp, rows, accsh, gsem, ssem,
              cid, sid, sb)
    _msg_pass(htb_hbm, outb_hbm, srcp, dstp, wp, rows, accsh, gsem, ssem,
              cid, sid, sb)


def _msg2_body(src_hbm, dst_hbm, w_hbm, ht_hbm, out_hbm,
               srcp, dstp, wp, rows, accsh, *sems):
    gsem, ssem = sems[:NBUF], sems[NBUF:]
    cid = lax.axis_index("c")
    sid = lax.axis_index("s")
    wid = cid * NS + sid
    sb = _slab_base(sid)
    pltpu.sync_copy(src_hbm.at[pl.ds(wid * EPW, EPW)], srcp)
    pltpu.sync_copy(dst_hbm.at[pl.ds(wid * EPW, EPW)], dstp)
    pltpu.sync_copy(w_hbm.at[pl.ds(wid * EPW, EPW)], wp)
    _msg_pass(ht_hbm, out_hbm, srcp, dstp, wp, rows, accsh, gsem, ssem,
              cid, sid, sb)


_MSG_SCRATCH = [
    pltpu.VMEM((EPW,), jnp.int32),
    pltpu.VMEM((EPW,), jnp.int32),
    pltpu.VMEM((EPW,), jnp.float32),
    pltpu.VMEM((NBUF, KB, FH), jnp.float32),
    pltpu.VMEM_SHARED((N, FH), jnp.float32),
] + [pltpu.SemaphoreType.DMA] * (2 * NBUF)

_msg_kernel_128 = pl.kernel(
    _msg1_body,
    out_type=[jax.ShapeDtypeStruct((2 * N, FH), jnp.float32),
              jax.ShapeDtypeStruct((2 * N, FH), jnp.float32)],
    mesh=_MESH,
    compiler_params=_SC_PARAMS,
    scratch_types=_MSG_SCRATCH,
)

_msg_kernel_64 = pl.kernel(
    _msg2_body,
    out_type=jax.ShapeDtypeStruct((2 * N, FH), jnp.float32),
    mesh=_MESH,
    compiler_params=_SC_PARAMS,
    scratch_types=_MSG_SCRATCH,
)


_R = 1000
_G = N // _R


def _tc1_body(x_ref, w_ref, dga_ref, dgb_ref, hta_ref, htb_ref, dis_ref):
    dis = lax.rsqrt(dga_ref[:, :1] + dgb_ref[:, :1] + 1.0)
    ht = jnp.dot(x_ref[...], w_ref[...],
                 preferred_element_type=jnp.float32) * dis
    hta_ref[...] = ht[:, :FH]
    htb_ref[...] = ht[:, FH:]
    dis_ref[...] = dis


def _tc1(x, W1, deg_parts):
    return pl.pallas_call(
        _tc1_body,
        grid=(_G,),
        in_specs=[
            pl.BlockSpec((_R, 128), lambda i: (i, 0)),
            pl.BlockSpec((128, 128), lambda i: (0, 0)),
            pl.BlockSpec((_R, 16), lambda i: (i, 0)),
            pl.BlockSpec((_R, 16), lambda i: (i + _G, 0)),
        ],
        out_specs=[
            pl.BlockSpec((_R, FH), lambda i: (i, 0)),
            pl.BlockSpec((_R, FH), lambda i: (i, 0)),
            pl.BlockSpec((_R, 1), lambda i: (i, 0)),
        ],
        out_shape=[
            jax.ShapeDtypeStruct((N, FH), jnp.float32),
            jax.ShapeDtypeStruct((N, FH), jnp.float32),
            jax.ShapeDtypeStruct((N, 1), jnp.float32),
        ],
    )(x, W1, deg_parts, deg_parts)


def _tc2_body(a0a_ref, a1a_ref, a0b_ref, a1b_ref, hta_ref, htb_ref,
              dis_ref, b_ref, w_ref, out_ref):
    dis = dis_ref[...]
    xa = dis * (a0a_ref[...] + a1a_ref[...] + hta_ref[...])
    xb = dis * (a0b_ref[...] + a1b_ref[...] + htb_ref[...])
    x1 = jnp.concatenate([xa, xb], axis=1) + b_ref[...]
    out_ref[...] = jnp.dot(x1, w_ref[...],
                           preferred_element_type=jnp.float32) * dis


def _tc2(acc1a, acc1b, ht1a, ht1b, dis, b1, W2):
    return pl.pallas_call(
        _tc2_body,
        grid=(_G,),
        in_specs=[
            pl.BlockSpec((_R, FH), lambda i: (i, 0)),
            pl.BlockSpec((_R, FH), lambda i: (i + _G, 0)),
            pl.BlockSpec((_R, FH), lambda i: (i, 0)),
            pl.BlockSpec((_R, FH), lambda i: (i + _G, 0)),
            pl.BlockSpec((_R, FH), lambda i: (i, 0)),
            pl.BlockSpec((_R, FH), lambda i: (i, 0)),
            pl.BlockSpec((_R, 1), lambda i: (i, 0)),
            pl.BlockSpec((1, 128), lambda i: (0, 0)),
            pl.BlockSpec((128, 64), lambda i: (0, 0)),
        ],
        out_specs=pl.BlockSpec((_R, 64), lambda i: (i, 0)),
        out_shape=jax.ShapeDtypeStruct((N, 64), jnp.float32),
    )(acc1a, acc1a, acc1b, acc1b, ht1a, ht1b, dis, b1, W2)


def _tc3_body(a0_ref, a1_ref, ht_ref, dis_ref, b_ref, out_ref):
    out_ref[...] = (dis_ref[...] * (a0_ref[...] + a1_ref[...] + ht_ref[...])
                    + b_ref[...])


def _tc3(acc2, ht2, dis, b2):
    return pl.pallas_call(
        _tc3_body,
        grid=(_G,),
        in_specs=[
            pl.BlockSpec((_R, 64), lambda i: (i, 0)),
            pl.BlockSpec((_R, 64), lambda i: (i + _G, 0)),
            pl.BlockSpec((_R, 64), lambda i: (i, 0)),
            pl.BlockSpec((_R, 1), lambda i: (i, 0)),
            pl.BlockSpec((1, 64), lambda i: (0, 0)),
        ],
        out_specs=pl.BlockSpec((_R, 64), lambda i: (i, 0)),
        out_shape=jax.ShapeDtypeStruct((N, 64), jnp.float32),
    )(acc2, acc2, ht2, dis, b2)


def kernel(x, edge_index, edge_weight, W1, b1, W2, b2):
    src = edge_index[0].astype(jnp.int32)
    dst = edge_index[1].astype(jnp.int32)
    w = edge_weight.astype(jnp.float32)
    z16 = jnp.zeros((N, 16), jnp.float32)

    deg_parts = _deg_kernel(dst, w, z16)
    ht1a, ht1b, dis = _tc1(x, W1, deg_parts)
    acc1a, acc1b = _msg_kernel_128(src, dst, w, ht1a, ht1b)
    ht2 = _tc2(acc1a, acc1b, ht1a, ht1b, dis,
               b1.reshape(1, -1), W2)
    acc2 = _msg_kernel_64(src, dst, w, ht2)
    return _tc3(acc2, ht2, dis, b2.reshape(1, -1))


# speedup vs baseline: 3.4499x; 2.1331x over previous
import functools

import jax
import jax.numpy as jnp
from jax import lax
from jax.experimental import pallas as pl
from jax.experimental.pallas import tpu as pltpu
from jax.experimental.pallas import tpu_sc as plsc

N = 10000
E = 320000
NC = 2
NS = 16
NW = NC * NS
EPW = E // NW
KB = 40
NB = EPW // KB
KBD = 80
NBD = EPW // KBD
SLAB = 632


def _slab_base(sid):
    return jnp.minimum(sid * SLAB, N - SLAB)

_MESH = plsc.VectorSubcoreMesh(core_axis_name="c", subcore_axis_name="s")
_SC_PARAMS = pltpu.CompilerParams(use_tc_tiling_on_sc=False)


NBUF = 5
NR = NB // NBUF
NRD = NBD // NBUF


def _deg_body(dst_hbm, w_hbm, z_hbm, deg_hbm, dstp, wp, rows, degsh, *sems):
    cid = lax.axis_index("c")
    sid = lax.axis_index("s")
    wid = cid * NS + sid
    sb = _slab_base(sid)
    pltpu.sync_copy(z_hbm.at[pl.ds(sb, SLAB)], degsh.at[pl.ds(sb, SLAB)])
    pltpu.sync_copy(dst_hbm.at[pl.ds(wid * EPW, EPW)], dstp)
    pltpu.sync_copy(w_hbm.at[pl.ds(wid * EPW, EPW)], wp)
    plsc.subcore_barrier()

    def round_(r, carry):
        base = r * NBUF
        for b in range(NBUF):
            e0 = (base + b) * KBD
            for g16 in range(KBD // 16):
                wv = wp[pl.ds(e0 + g16 * 16, 16)]
                for j in range(16):
                    rows[b, g16 * 16 + j, :] = jnp.broadcast_to(wv[j], (16,))
            pltpu.async_copy(rows.at[b], degsh.at[dstp.at[pl.ds(e0, KBD)]],
                             sems[b], add=True)
        for b in range(NBUF):
            e0 = (base + b) * KBD
            pltpu.make_async_copy(rows.at[b],
                                  degsh.at[dstp.at[pl.ds(e0, KBD)]],
                                  sems[b]).wait()
        return carry

    lax.fori_loop(0, NRD, round_, 0)
    plsc.subcore_barrier()
    pltpu.sync_copy(degsh.at[pl.ds(sb, SLAB)],
                    deg_hbm.at[pl.ds(cid * N + sb, SLAB)])


_deg_kernel = pl.kernel(
    _deg_body,
    out_type=jax.ShapeDtypeStruct((2 * N, 16), jnp.float32),
    mesh=_MESH,
    compiler_params=_SC_PARAMS,
    scratch_types=[
        pltpu.VMEM((EPW,), jnp.int32),
        pltpu.VMEM((EPW,), jnp.float32),
        pltpu.VMEM((NBUF, KBD, 16), jnp.float32),
        pltpu.VMEM_SHARED((N, 16), jnp.float32),
    ] + [pltpu.SemaphoreType.DMA] * NBUF,
)


FH = 64


def _msg_pass(ht_hbm, out_hbm, srcp, dstp, wp, rows,
              accsh, gsem, ssem, cid, sid, sb):
    def zrow(k, c):
        for f in range(FH // 16):
            rows[0, 0, k, pl.ds(f * 16, 16)] = jnp.zeros((16,), jnp.float32)
        return c

    lax.fori_loop(0, KB, zrow, 0)
    for j in range(SLAB // KB):
        pltpu.sync_copy(rows.at[0, 0], accsh.at[pl.ds(sb + j * KB, KB)])
    _tail = SLAB % KB
    pltpu.sync_copy(rows.at[0, 0, pl.ds(0, _tail)],
                    accsh.at[pl.ds(sb + (SLAB // KB) * KB, _tail)])
    plsc.subcore_barrier()

    def _fire(g, s, b):
        pltpu.async_copy(ht_hbm.at[srcp.at[pl.ds(g * KB, KB)]],
                         rows.at[s, b], gsem[s * NBUF + b])

    def _wait_fire(g, s, b):
        pltpu.make_async_copy(ht_hbm.at[srcp.at[pl.ds(g * KB, KB)]],
                              rows.at[s, b], gsem[s * NBUF + b]).wait()

    def _scale(g, s, b):
        e0 = g * KB

        def scale16(g16, c):
            wv = wp[pl.ds(e0 + g16 * 16, 16)]
            for j in range(16):
                k = g16 * 16 + j
                sc = wv[j]
                for f in range(FH // 16):
                    sl = pl.ds(f * 16, 16)
                    rows[s, b, k, sl] = rows[s, b, k, sl] * sc
            return c

        lax.fori_loop(0, KB // 16, scale16, 0)
        wv = wp[pl.ds(e0 + KB - 16, 16)]
        for j in range(16 - (KB - (KB // 16) * 16), 16):
            k = KB - 16 + j
            sc = wv[j]
            for f in range(FH // 16):
                sl = pl.ds(f * 16, 16)
                rows[s, b, k, sl] = rows[s, b, k, sl] * sc

    def _fire_scat(g, s, b):
        pltpu.async_copy(rows.at[s, b], accsh.at[dstp.at[pl.ds(g * KB, KB)]],
                         ssem[s * NBUF + b], add=True)

    def _wait_scat(g, s, b):
        pltpu.make_async_copy(rows.at[s, b],
                              accsh.at[dstp.at[pl.ds(g * KB, KB)]],
                              ssem[s * NBUF + b]).wait()

    def _half(r, S, T, first):
        for b in range(NBUF):
            g = r * NBUF + b
            _wait_fire(g, S, b)
            _scale(g, S, b)
            if first:
                _fire(g + NBUF, T, b)
            else:
                @pl.when(r < NR - 1)
                def _():
                    _wait_scat(g, T, b)
                    _fire(g + NBUF, T, b)
            _fire_scat(g, S, b)

    for b in range(NBUF):
        _fire(b, 0, b)
    _half(0, 0, 1, True)

    def pair(k, carry):
        _half(2 * k - 1, 1, 0, False)
        _half(2 * k, 0, 1, False)
        return carry

    lax.fori_loop(1, NR // 2, pair, 0)
    _half(NR - 1, 1, 0, False)
    for b in range(NBUF):
        _wait_scat(0, 0, b)
        _wait_scat(0, 1, b)
    plsc.subcore_barrier()
    pltpu.sync_copy(accsh.at[pl.ds(sb, SLAB)],
                    out_hbm.at[pl.ds(cid * N + sb, SLAB)])


def _msg1_body(src_hbm, dst_hbm, w_hbm, hta_hbm, htb_hbm, outa_hbm, outb_hbm,
               srcp, dstp, wp, rows, accsh, *sems):
    gsem, ssem = sems[:2 * NBUF], sems[2 * NBUF:]
    cid = lax.axis_index("c")
    sid = lax.axis_index("s")
    wid = cid * NS + sid
    sb = _slab_base(sid)
    ebase = wid * EPW
    pltpu.sync_copy(src_hbm.at[pl.ds(ebase, EPW)], srcp)
    pltpu.sync_copy(dst_hbm.at[pl.ds(ebase, EPW)], dstp)
    pltpu.sync_copy(w_hbm.at[pl.ds(ebase, EPW)], wp)
    _msg_pass(hta_hbm, outa_hbm, srcp, dstp, wp,
              rows, accsh, gsem, ssem, cid, sid, sb)
    _msg_pass(htb_hbm, outb_hbm, srcp, dstp, wp,
              rows, accsh, gsem, ssem, cid, sid, sb)


def _msg2_body(src_hbm, dst_hbm, w_hbm, ht_hbm, out_hbm,
               srcp, dstp, wp, rows, accsh, *sems):
    gsem, ssem = sems[:2 * NBUF], sems[2 * NBUF:]
    cid = lax.axis_index("c")
    sid = lax.axis_index("s")
    wid = cid * NS + sid
    sb = _slab_base(sid)
    ebase = wid * EPW
    pltpu.sync_copy(src_hbm.at[pl.ds(ebase, EPW)], srcp)
    pltpu.sync_copy(dst_hbm.at[pl.ds(ebase, EPW)], dstp)
    pltpu.sync_copy(w_hbm.at[pl.ds(ebase, EPW)], wp)
    _msg_pass(ht_hbm, out_hbm, srcp, dstp, wp,
              rows, accsh, gsem, ssem, cid, sid, sb)


_MSG_SCRATCH = [
    pltpu.VMEM((EPW,), jnp.int32),
    pltpu.VMEM((EPW,), jnp.int32),
    pltpu.VMEM((EPW,), jnp.float32),
    pltpu.VMEM((2, NBUF, KB, FH), jnp.float32),
    pltpu.VMEM_SHARED((N, FH), jnp.float32),
] + [pltpu.SemaphoreType.DMA] * (4 * NBUF)

_msg_kernel_128 = pl.kernel(
    _msg1_body,
    out_type=[jax.ShapeDtypeStruct((2 * N, FH), jnp.float32),
              jax.ShapeDtypeStruct((2 * N, FH), jnp.float32)],
    mesh=_MESH,
    compiler_params=_SC_PARAMS,
    scratch_types=_MSG_SCRATCH,
)

_msg_kernel_64 = pl.kernel(
    _msg2_body,
    out_type=jax.ShapeDtypeStruct((2 * N, FH), jnp.float32),
    mesh=_MESH,
    compiler_params=_SC_PARAMS,
    scratch_types=_MSG_SCRATCH,
)


_R = 1000
_G = N // _R


def _tc1_body(x_ref, w_ref, dga_ref, dgb_ref, hta_ref, htb_ref, dis_ref):
    dis = lax.rsqrt(dga_ref[:, :1] + dgb_ref[:, :1] + 1.0)
    ht = jnp.dot(x_ref[...], w_ref[...],
                 preferred_element_type=jnp.float32) * dis
    hta_ref[...] = ht[:, :FH]
    htb_ref[...] = ht[:, FH:]
    dis_ref[...] = dis


def _tc1(x, W1, deg_parts):
    return pl.pallas_call(
        _tc1_body,
        grid=(_G,),
        in_specs=[
            pl.BlockSpec((_R, 128), lambda i: (i, 0)),
            pl.BlockSpec((128, 128), lambda i: (0, 0)),
            pl.BlockSpec((_R, 16), lambda i: (i, 0)),
            pl.BlockSpec((_R, 16), lambda i: (i + _G, 0)),
        ],
        out_specs=[
            pl.BlockSpec((_R, FH), lambda i: (i, 0)),
            pl.BlockSpec((_R, FH), lambda i: (i, 0)),
            pl.BlockSpec((_R, 1), lambda i: (i, 0)),
        ],
        out_shape=[
            jax.ShapeDtypeStruct((N, FH), jnp.float32),
            jax.ShapeDtypeStruct((N, FH), jnp.float32),
            jax.ShapeDtypeStruct((N, 1), jnp.float32),
        ],
    )(x, W1, deg_parts, deg_parts)


def _tc2_body(a0a_ref, a1a_ref, a0b_ref, a1b_ref, hta_ref, htb_ref,
              dis_ref, b_ref, w_ref, out_ref):
    dis = dis_ref[...]
    xa = dis * (a0a_ref[...] + a1a_ref[...] + hta_ref[...])
    xb = dis * (a0b_ref[...] + a1b_ref[...] + htb_ref[...])
    x1 = jnp.concatenate([xa, xb], axis=1) + b_ref[...]
    out_ref[...] = jnp.dot(x1, w_ref[...],
                           preferred_element_type=jnp.float32) * dis


def _tc2(acc1a, acc1b, ht1a, ht1b, dis, b1, W2):
    return pl.pallas_call(
        _tc2_body,
        grid=(_G,),
        in_specs=[
            pl.BlockSpec((_R, FH), lambda i: (i, 0)),
            pl.BlockSpec((_R, FH), lambda i: (i + _G, 0)),
            pl.BlockSpec((_R, FH), lambda i: (i, 0)),
            pl.BlockSpec((_R, FH), lambda i: (i + _G, 0)),
            pl.BlockSpec((_R, FH), lambda i: (i, 0)),
            pl.BlockSpec((_R, FH), lambda i: (i, 0)),
            pl.BlockSpec((_R, 1), lambda i: (i, 0)),
            pl.BlockSpec((1, 128), lambda i: (0, 0)),
            pl.BlockSpec((128, 64), lambda i: (0, 0)),
        ],
        out_specs=pl.BlockSpec((_R, 64), lambda i: (i, 0)),
        out_shape=jax.ShapeDtypeStruct((N, 64), jnp.float32),
    )(acc1a, acc1a, acc1b, acc1b, ht1a, ht1b, dis, b1, W2)


def _tc3_body(a0_ref, a1_ref, ht_ref, dis_ref, b_ref, out_ref):
    out_ref[...] = (dis_ref[...] * (a0_ref[...] + a1_ref[...] + ht_ref[...])
                    + b_ref[...])


def _tc3(acc2, ht2, dis, b2):
    return pl.pallas_call(
        _tc3_body,
        grid=(_G,),
        in_specs=[
            pl.BlockSpec((_R, 64), lambda i: (i, 0)),
            pl.BlockSpec((_R, 64), lambda i: (i + _G, 0)),
            pl.BlockSpec((_R, 64), lambda i: (i, 0)),
            pl.BlockSpec((_R, 1), lambda i: (i, 0)),
            pl.BlockSpec((1, 64), lambda i: (0, 0)),
        ],
        out_specs=pl.BlockSpec((_R, 64), lambda i: (i, 0)),
        out_shape=jax.ShapeDtypeStruct((N, 64), jnp.float32),
    )(acc2, acc2, ht2, dis, b2)


def kernel(x, edge_index, edge_weight, W1, b1, W2, b2):
    src = edge_index[0].astype(jnp.int32)
    dst = edge_index[1].astype(jnp.int32)
    w = edge_weight.astype(jnp.float32)
    z16 = jnp.zeros((N, 16), jnp.float32)

    deg_parts = _deg_kernel(dst, w, z16)
    ht1a, ht1b, dis = _tc1(x, W1, deg_parts)
    acc1a, acc1b = _msg_kernel_128(src, dst, w, ht1a, ht1b)
    ht2 = _tc2(acc1a, acc1b, ht1a, ht1b, dis,
               b1.reshape(1, -1), W2)
    acc2 = _msg_kernel_64(src, dst, w, ht2)
    return _tc3(acc2, ht2, dis, b2.reshape(1, -1))


# speedup vs baseline: 3.5053x; 1.0160x over previous
import functools

import jax
import jax.numpy as jnp
from jax import lax
from jax.experimental import pallas as pl
from jax.experimental.pallas import tpu as pltpu
from jax.experimental.pallas import tpu_sc as plsc

N = 10000
E = 320000
NC = 2
NS = 16
NW = NC * NS
EPW = E // NW
KB = 40
NB = EPW // KB
KBD = 80
NBD = EPW // KBD
SLAB = 632


def _slab_base(sid):
    return jnp.minimum(sid * SLAB, N - SLAB)

_MESH = plsc.VectorSubcoreMesh(core_axis_name="c", subcore_axis_name="s")
_SC_PARAMS = pltpu.CompilerParams(use_tc_tiling_on_sc=False)


NBUF = 5
NR = NB // NBUF
NRD = NBD // NBUF


def _deg_body(dst_hbm, w_hbm, deg_hbm, dstp, wp, rows, degsh, *sems):
    cid = lax.axis_index("c")
    sid = lax.axis_index("s")
    wid = cid * NS + sid
    sb = _slab_base(sid)
    def zrow(k, c):
        rows[0, k, :] = jnp.zeros((16,), jnp.float32)
        return c

    lax.fori_loop(0, KBD, zrow, 0)
    for j in range(SLAB // KBD):
        pltpu.sync_copy(rows.at[0], degsh.at[pl.ds(sb + j * KBD, KBD)])
    pltpu.sync_copy(rows.at[0, pl.ds(0, SLAB % KBD)],
                    degsh.at[pl.ds(sb + (SLAB // KBD) * KBD, SLAB % KBD)])
    pltpu.sync_copy(dst_hbm.at[pl.ds(wid * EPW, EPW)], dstp)
    pltpu.sync_copy(w_hbm.at[pl.ds(wid * EPW, EPW)], wp)
    plsc.subcore_barrier()

    def round_(r, carry):
        base = r * NBUF
        for b in range(NBUF):
            e0 = (base + b) * KBD
            for g16 in range(KBD // 16):
                wv = wp[pl.ds(e0 + g16 * 16, 16)]
                for j in range(16):
                    rows[b, g16 * 16 + j, :] = jnp.broadcast_to(wv[j], (16,))
            pltpu.async_copy(rows.at[b], degsh.at[dstp.at[pl.ds(e0, KBD)]],
                             sems[b], add=True)
        for b in range(NBUF):
            e0 = (base + b) * KBD
            pltpu.make_async_copy(rows.at[b],
                                  degsh.at[dstp.at[pl.ds(e0, KBD)]],
                                  sems[b]).wait()
        return carry

    lax.fori_loop(0, NRD, round_, 0)
    plsc.subcore_barrier()
    pltpu.sync_copy(degsh.at[pl.ds(sb, SLAB)],
                    deg_hbm.at[pl.ds(cid * N + sb, SLAB)])


_deg_kernel = pl.kernel(
    _deg_body,
    out_type=jax.ShapeDtypeStruct((2 * N, 16), jnp.float32),
    mesh=_MESH,
    compiler_params=_SC_PARAMS,
    scratch_types=[
        pltpu.VMEM((EPW,), jnp.int32),
        pltpu.VMEM((EPW,), jnp.float32),
        pltpu.VMEM((NBUF, KBD, 16), jnp.float32),
        pltpu.VMEM_SHARED((N, 16), jnp.float32),
    ] + [pltpu.SemaphoreType.DMA] * NBUF,
)


FH = 64


def _msg_pass(ht_hbm, out_hbm, srcp, dstp, wp, rows,
              accsh, gsem, ssem, cid, sid, sb):
    def zrow(k, c):
        for f in range(FH // 16):
            rows[0, 0, k, pl.ds(f * 16, 16)] = jnp.zeros((16,), jnp.float32)
        return c

    lax.fori_loop(0, KB, zrow, 0)
    for j in range(SLAB // KB):
        pltpu.sync_copy(rows.at[0, 0], accsh.at[pl.ds(sb + j * KB, KB)])
    _tail = SLAB % KB
    pltpu.sync_copy(rows.at[0, 0, pl.ds(0, _tail)],
                    accsh.at[pl.ds(sb + (SLAB // KB) * KB, _tail)])
    plsc.subcore_barrier()

    def _fire(g, s, b):
        pltpu.async_copy(ht_hbm.at[srcp.at[pl.ds(g * KB, KB)]],
                         rows.at[s, b], gsem[s * NBUF + b])

    def _wait_fire(g, s, b):
        pltpu.make_async_copy(ht_hbm.at[srcp.at[pl.ds(g * KB, KB)]],
                              rows.at[s, b], gsem[s * NBUF + b]).wait()

    def _scale(g, s, b):
        e0 = g * KB

        def scale16(g16, c):
            wv = wp[pl.ds(e0 + g16 * 16, 16)]
            for j in range(16):
                k = g16 * 16 + j
                sc = wv[j]
                for f in range(FH // 16):
                    sl = pl.ds(f * 16, 16)
                    rows[s, b, k, sl] = rows[s, b, k, sl] * sc
            return c

        lax.fori_loop(0, KB // 16, scale16, 0)
        wv = wp[pl.ds(e0 + KB - 16, 16)]
        for j in range(16 - (KB - (KB // 16) * 16), 16):
            k = KB - 16 + j
            sc = wv[j]
            for f in range(FH // 16):
                sl = pl.ds(f * 16, 16)
                rows[s, b, k, sl] = rows[s, b, k, sl] * sc

    def _fire_scat(g, s, b):
        pltpu.async_copy(rows.at[s, b], accsh.at[dstp.at[pl.ds(g * KB, KB)]],
                         ssem[s * NBUF + b], add=True)

    def _wait_scat(g, s, b):
        pltpu.make_async_copy(rows.at[s, b],
                              accsh.at[dstp.at[pl.ds(g * KB, KB)]],
                              ssem[s * NBUF + b]).wait()

    def _half(r, S, T, first):
        for b in range(NBUF):
            g = r * NBUF + b
            _wait_fire(g, S, b)
            _scale(g, S, b)
            if first:
                _fire(g + NBUF, T, b)
            else:
                @pl.when(r < NR - 1)
                def _():
                    _wait_scat(g, T, b)
                    _fire(g + NBUF, T, b)
            _fire_scat(g, S, b)

    for b in range(NBUF):
        _fire(b, 0, b)
    _half(0, 0, 1, True)

    def pair(k, carry):
        _half(2 * k - 1, 1, 0, False)
        _half(2 * k, 0, 1, False)
        return carry

    lax.fori_loop(1, NR // 2, pair, 0)
    _half(NR - 1, 1, 0, False)
    for b in range(NBUF):
        _wait_scat(0, 0, b)
        _wait_scat(0, 1, b)
    plsc.subcore_barrier()
    pltpu.sync_copy(accsh.at[pl.ds(sb, SLAB)],
                    out_hbm.at[pl.ds(cid * N + sb, SLAB)])


def _msg1_body(src_hbm, dst_hbm, w_hbm, hta_hbm, htb_hbm, outa_hbm, outb_hbm,
               srcp, dstp, wp, rows, accsh, *sems):
    gsem, ssem = sems[:2 * NBUF], sems[2 * NBUF:]
    cid = lax.axis_index("c")
    sid = lax.axis_index("s")
    wid = cid * NS + sid
    sb = _slab_base(sid)
    ebase = wid * EPW
    pltpu.sync_copy(src_hbm.at[pl.ds(ebase, EPW)], srcp)
    pltpu.sync_copy(dst_hbm.at[pl.ds(ebase, EPW)], dstp)
    pltpu.sync_copy(w_hbm.at[pl.ds(ebase, EPW)], wp)
    _msg_pass(hta_hbm, outa_hbm, srcp, dstp, wp,
              rows, accsh, gsem, ssem, cid, sid, sb)
    _msg_pass(htb_hbm, outb_hbm, srcp, dstp, wp,
              rows, accsh, gsem, ssem, cid, sid, sb)


def _msg2_body(src_hbm, dst_hbm, w_hbm, ht_hbm, out_hbm,
               srcp, dstp, wp, rows, accsh, *sems):
    gsem, ssem = sems[:2 * NBUF], sems[2 * NBUF:]
    cid = lax.axis_index("c")
    sid = lax.axis_index("s")
    wid = cid * NS + sid
    sb = _slab_base(sid)
    ebase = wid * EPW
    pltpu.sync_copy(src_hbm.at[pl.ds(ebase, EPW)], srcp)
    pltpu.sync_copy(dst_hbm.at[pl.ds(ebase, EPW)], dstp)
    pltpu.sync_copy(w_hbm.at[pl.ds(ebase, EPW)], wp)
    _msg_pass(ht_hbm, out_hbm, srcp, dstp, wp,
              rows, accsh, gsem, ssem, cid, sid, sb)


_MSG_SCRATCH = [
    pltpu.VMEM((EPW,), jnp.int32),
    pltpu.VMEM((EPW,), jnp.int32),
    pltpu.VMEM((EPW,), jnp.float32),
    pltpu.VMEM((2, NBUF, KB, FH), jnp.float32),
    pltpu.VMEM_SHARED((N, FH), jnp.float32),
] + [pltpu.SemaphoreType.DMA] * (4 * NBUF)

_msg_kernel_128 = pl.kernel(
    _msg1_body,
    out_type=[jax.ShapeDtypeStruct((2 * N, FH), jnp.float32),
              jax.ShapeDtypeStruct((2 * N, FH), jnp.float32)],
    mesh=_MESH,
    compiler_params=_SC_PARAMS,
    scratch_types=_MSG_SCRATCH,
)

_msg_kernel_64 = pl.kernel(
    _msg2_body,
    out_type=jax.ShapeDtypeStruct((2 * N, FH), jnp.float32),
    mesh=_MESH,
    compiler_params=_SC_PARAMS,
    scratch_types=_MSG_SCRATCH,
)


_R = N
_G = N // _R


def _tc1_body(x_ref, w_ref, dga_ref, dgb_ref, hta_ref, htb_ref, dis_ref):
    dis = lax.rsqrt(dga_ref[:, :1] + dgb_ref[:, :1] + 1.0)
    ht = jnp.dot(x_ref[...], w_ref[...],
                 preferred_element_type=jnp.float32) * dis
    hta_ref[...] = ht[:, :FH]
    htb_ref[...] = ht[:, FH:]
    dis_ref[...] = dis


def _tc1(x, W1, deg_parts):
    return pl.pallas_call(
        _tc1_body,
        grid=(_G,),
        in_specs=[
            pl.BlockSpec((_R, 128), lambda i: (i, 0)),
            pl.BlockSpec((128, 128), lambda i: (0, 0)),
            pl.BlockSpec((_R, 16), lambda i: (i, 0)),
            pl.BlockSpec((_R, 16), lambda i: (i + _G, 0)),
        ],
        out_specs=[
            pl.BlockSpec((_R, FH), lambda i: (i, 0)),
            pl.BlockSpec((_R, FH), lambda i: (i, 0)),
            pl.BlockSpec((_R, 1), lambda i: (i, 0)),
        ],
        out_shape=[
            jax.ShapeDtypeStruct((N, FH), jnp.float32),
            jax.ShapeDtypeStruct((N, FH), jnp.float32),
            jax.ShapeDtypeStruct((N, 1), jnp.float32),
        ],
    )(x, W1, deg_parts, deg_parts)


def _tc2_body(a0a_ref, a1a_ref, a0b_ref, a1b_ref, hta_ref, htb_ref,
              dis_ref, b_ref, w_ref, out_ref):
    dis = dis_ref[...]
    xa = dis * (a0a_ref[...] + a1a_ref[...] + hta_ref[...])
    xb = dis * (a0b_ref[...] + a1b_ref[...] + htb_ref[...])
    x1 = jnp.concatenate([xa, xb], axis=1) + b_ref[...]
    out_ref[...] = jnp.dot(x1, w_ref[...],
                           preferred_element_type=jnp.float32) * dis


def _tc2(acc1a, acc1b, ht1a, ht1b, dis, b1, W2):
    return pl.pallas_call(
        _tc2_body,
        grid=(_G,),
        in_specs=[
            pl.BlockSpec((_R, FH), lambda i: (i, 0)),
            pl.BlockSpec((_R, FH), lambda i: (i + _G, 0)),
            pl.BlockSpec((_R, FH), lambda i: (i, 0)),
            pl.BlockSpec((_R, FH), lambda i: (i + _G, 0)),
            pl.BlockSpec((_R, FH), lambda i: (i, 0)),
            pl.BlockSpec((_R, FH), lambda i: (i, 0)),
            pl.BlockSpec((_R, 1), lambda i: (i, 0)),
            pl.BlockSpec((1, 128), lambda i: (0, 0)),
            pl.BlockSpec((128, 64), lambda i: (0, 0)),
        ],
        out_specs=pl.BlockSpec((_R, 64), lambda i: (i, 0)),
        out_shape=jax.ShapeDtypeStruct((N, 64), jnp.float32),
    )(acc1a, acc1a, acc1b, acc1b, ht1a, ht1b, dis, b1, W2)


def _tc3_body(a0_ref, a1_ref, ht_ref, dis_ref, b_ref, out_ref):
    out_ref[...] = (dis_ref[...] * (a0_ref[...] + a1_ref[...] + ht_ref[...])
                    + b_ref[...])


def _tc3(acc2, ht2, dis, b2):
    return pl.pallas_call(
        _tc3_body,
        grid=(_G,),
        in_specs=[
            pl.BlockSpec((_R, 64), lambda i: (i, 0)),
            pl.BlockSpec((_R, 64), lambda i: (i + _G, 0)),
            pl.BlockSpec((_R, 64), lambda i: (i, 0)),
            pl.BlockSpec((_R, 1), lambda i: (i, 0)),
            pl.BlockSpec((1, 64), lambda i: (0, 0)),
        ],
        out_specs=pl.BlockSpec((_R, 64), lambda i: (i, 0)),
        out_shape=jax.ShapeDtypeStruct((N, 64), jnp.float32),
    )(acc2, acc2, ht2, dis, b2)


def kernel(x, edge_index, edge_weight, W1, b1, W2, b2):
    src = edge_index[0].astype(jnp.int32)
    dst = edge_index[1].astype(jnp.int32)
    w = edge_weight.astype(jnp.float32)
    deg_parts = _deg_kernel(dst, w)
    ht1a, ht1b, dis = _tc1(x, W1, deg_parts)
    acc1a, acc1b = _msg_kernel_128(src, dst, w, ht1a, ht1b)
    ht2 = _tc2(acc1a, acc1b, ht1a, ht1b, dis,
               b1.reshape(1, -1), W2)
    acc2 = _msg_kernel_64(src, dst, w, ht2)
    return _tc3(acc2, ht2, dis, b2.reshape(1, -1))
